# baseline (speedup 1.0000x reference)
import functools

import jax
import jax.numpy as jnp
from jax import lax
from jax.experimental import pallas as pl
from jax.experimental.pallas import tpu as pltpu
from jax.experimental.pallas import tpu_sc as plsc

NC, NS = 2, 16
NW = NC * NS
EPS = 1e-5


SC_V = 983040


def _sc_reformat(embT, v_pad):
    k, v = embT.shape
    vch = 1024
    per_w = SC_V // NW
    nch = per_w // vch
    out_rows = v_pad * k // 128
    orpc = vch * k // 128

    mesh = plsc.VectorSubcoreMesh(core_axis_name="c", subcore_axis_name="s")

    @functools.partial(
        pl.kernel,
        mesh=mesh,
        compiler_params=pltpu.CompilerParams(use_tc_tiling_on_sc=True,
                                             needs_layout_passes=False),
        out_type=jax.ShapeDtypeStruct((out_rows, 128), jnp.float32),
        scratch_types=[
            pltpu.VMEM((2, k, vch), jnp.float32),
            pltpu.VMEM((2, orpc, 128), jnp.float32),
            pltpu.SemaphoreType.DMA((2,)),
            pltpu.SemaphoreType.DMA((2,)),
        ],
    )
    def refmt_kernel(embT_hbm, out_hbm, slab, ob, isem, osem):
        wid = lax.axis_index("s") * NC + lax.axis_index("c")
        vbase = wid * per_w
        row_iota = lax.iota(jnp.int32, 16)
        in_pending = [None, None]
        out_pending = [None, None]
        in_pending[0] = pltpu.async_copy(
            embT_hbm.at[:, pl.ds(pl.multiple_of(vbase, 128), vch)],
            slab.at[0], isem.at[0])
        for ch in range(nch):
            cur = ch % 2
            nxt = (ch + 1) % 2
            if ch + 1 < nch:
                in_pending[nxt] = pltpu.async_copy(
                    embT_hbm.at[:, pl.ds(
                        pl.multiple_of(vbase + (ch + 1) * vch, 128), vch)],
                    slab.at[nxt], isem.at[nxt])
            in_pending[cur].wait()
            if out_pending[cur] is not None:
                out_pending[cur].wait()
            sl = slab.at[cur]
            obuf = ob.at[cur]

            def shuffle(jv, _, sl=sl, obuf=obuf):
                rowsplat = jnp.full((16,), jv, jnp.int32)
                colbase = row_iota * 0 + jv * 8
                for u in range(8):
                    col = colbase + u
                    val = plsc.load_gather(sl, [row_iota, col])
                    plsc.store_scatter(obuf, [rowsplat, row_iota + u * 16],
                                       val)
                return 0

            lax.fori_loop(0, vch // 8, shuffle, 0)
            out_pending[cur] = pltpu.async_copy(
                obuf,
                out_hbm.at[pl.ds(
                    pl.multiple_of((vbase + ch * vch) * k // 128, 128), orpc)],
                osem.at[cur])
        for p in out_pending:
            if p is not None:
                p.wait()

    return refmt_kernel(embT)


def _tc_tail_reformat(embT, bulk, v_pad):
    k, v = embT.shape
    c = 16384
    base_blk = SC_V // c
    g = pl.cdiv(v_pad - SC_V, c)
    out_rows = v_pad * k // 128
    rblk = c * k // 128

    def body(t_ref, al_ref, o_ref):
        x = t_ref[...]
        xt3 = x.T.reshape(c // 64, 64, k)
        out3 = jnp.concatenate(
            [xt3[:, 8 * m:8 * m + 8, :] for m in range(8)], axis=2)
        o_ref[...] = out3.reshape(rblk, 128)

    return pl.pallas_call(
        body,
        grid=(g,),
        in_specs=[pl.BlockSpec((k, c), lambda i: (0, i + base_blk)),
                  pl.BlockSpec((8, 128), lambda i: (0, 0))],
        out_specs=pl.BlockSpec((rblk, 128), lambda i: (i + base_blk, 0)),
        out_shape=jax.ShapeDtypeStruct((out_rows, 128), jnp.float32),
        input_output_aliases={1: 0},
    )(embT, bulk)


def _swap_idx(idx):
    lo = idx & 63
    return (idx ^ lo) | ((lo & 7) << 3) | (lo >> 3)


def _sc_gather(idx_flat, idx_raw, emb_table, lin_flat, *, n_idx, k):
    per_w = n_idx // NW
    chunk = 1664
    nchunk = per_w // chunk
    assert per_w % chunk == 0 and per_w % 8 == 0 and chunk % 8 == 0

    mesh = plsc.VectorSubcoreMesh(core_axis_name="c", subcore_axis_name="s")

    @functools.partial(
        pl.kernel,
        mesh=mesh,
        compiler_params=pltpu.CompilerParams(use_tc_tiling_on_sc=False),
        out_type=(
            jax.ShapeDtypeStruct((n_idx, k), jnp.float32),
            jax.ShapeDtypeStruct((n_idx,), jnp.float32),
        ),
        scratch_types=[
            pltpu.VMEM((per_w,), jnp.int32),
            pltpu.VMEM((per_w,), jnp.int32),
            pltpu.VMEM((2, chunk, k), jnp.float32),
            pltpu.VMEM((2, chunk), jnp.float32),
            pltpu.SemaphoreType.DMA((2,)),
            pltpu.SemaphoreType.DMA((2,)),
        ],
    )
    def gather_kernel(idx_hbm, idx2_hbm, emb_hbm, lin_hbm, erows_out,
                      lvals_out, idx_v, idx2_v, ebuf, lbuf, gsem, osem):
        wid = lax.axis_index("s") * NC + lax.axis_index("c")
        base = wid * per_w
        pltpu.sync_copy(idx_hbm.at[pl.ds(base, per_w)], idx_v)
        pltpu.sync_copy(idx2_hbm.at[pl.ds(base, per_w)], idx2_v)
        pending = [None, None]
        for g in range(nchunk):
            bsl = g % 2
            if pending[bsl] is not None:
                for h in pending[bsl]:
                    h.wait()
            isl = idx_v.at[pl.ds(g * chunk, chunk)]
            isl2 = idx2_v.at[pl.ds(g * chunk, chunk)]
            h_e = pltpu.async_copy(emb_hbm.at[isl], ebuf.at[bsl], gsem.at[bsl])
            h_l = pltpu.async_copy(lin_hbm.at[isl2], lbuf.at[bsl], gsem.at[bsl])
            h_e.wait()
            h_l.wait()
            o_e = pltpu.async_copy(
                ebuf.at[bsl], erows_out.at[pl.ds(base + g * chunk, chunk)],
                osem.at[bsl])
            o_l = pltpu.async_copy(
                lbuf.at[bsl], lvals_out.at[pl.ds(base + g * chunk, chunk)],
                osem.at[bsl])
            pending[bsl] = (o_e, o_l)
        for p in pending:
            if p is not None:
                for h in p:
                    h.wait()

    return gather_kernel(idx_flat, idx_raw, emb_table, lin_flat)


def _tc_body(x_ref, lin_ref, a_ref, w1_ref, b1_ref, w2_ref, b2_ref,
             w3_ref, c_ref, out_ref):
    x = x_ref[...]
    s = jnp.dot(x, a_ref[...], preferred_element_type=jnp.float32)
    fm = 0.5 * (jnp.sum(s * s, axis=1, keepdims=True)
                - jnp.sum(x * x, axis=1, keepdims=True))
    lin = jnp.sum(lin_ref[...], axis=1, keepdims=True)
    h = jnp.dot(x, w1_ref[...], preferred_element_type=jnp.float32) + b1_ref[...]
    h = jnp.maximum(h, 0.0)
    h = jnp.dot(h, w2_ref[...], preferred_element_type=jnp.float32) + b2_ref[...]
    h = jnp.maximum(h, 0.0)
    mlp = jnp.dot(h, w3_ref[...], preferred_element_type=jnp.float32)
    out_ref[...] = fm + lin + mlp + c_ref[...]


def kernel(cat_features, emb_table, lin_table, bias,
           W1, b1, g1, be1, W2, b2, g2, be2, W3, b3):
    B, F = cat_features.shape
    V, K = emb_table.shape
    D_IN = F * K
    H = W1.shape[1]
    n_idx = B * F

    idx_raw = cat_features.reshape(-1)
    idx_flat = jnp.where(idx_raw < SC_V, idx_raw, _swap_idx(idx_raw))

    v_pad = (V + 63) // 64 * 64
    bulk = _sc_reformat(emb_table.T, v_pad)
    emb2d = _tc_tail_reformat(emb_table.T, bulk, v_pad)
    lin1d = lin_table.reshape(-1)
    emb_lin = emb2d.reshape(v_pad, K)

    erows, lvals = _sc_gather(idx_flat, idx_raw, emb_lin, lin1d,
                              n_idx=n_idx, k=K)
    x = erows.reshape(B, D_IN)
    lin_vals = lvals.reshape(B, F)

    s_bn = 1.0 / jnp.sqrt(1.0 + EPS)
    w1f = W1 * (g1 * s_bn)[None, :]
    b1f = (b1 * g1 * s_bn + be1)[None, :]
    w2f = W2 * (g2 * s_bn)[None, :]
    b2f = (b2 * g2 * s_bn + be2)[None, :]
    c = (bias + b3).reshape(1, 1)
    a_mat = jnp.tile(jnp.eye(K, dtype=jnp.float32), (F, 1))

    BS = 1024
    grid = (B // BS,)
    out = pl.pallas_call(
        _tc_body,
        grid=grid,
        in_specs=[
            pl.BlockSpec((BS, D_IN), lambda i: (i, 0)),
            pl.BlockSpec((BS, F), lambda i: (i, 0)),
            pl.BlockSpec((D_IN, K), lambda i: (0, 0)),
            pl.BlockSpec((D_IN, H), lambda i: (0, 0)),
            pl.BlockSpec((1, H), lambda i: (0, 0)),
            pl.BlockSpec((H, H), lambda i: (0, 0)),
            pl.BlockSpec((1, H), lambda i: (0, 0)),
            pl.BlockSpec((H, 1), lambda i: (0, 0)),
            pl.BlockSpec((1, 1), lambda i: (0, 0)),
        ],
        out_specs=pl.BlockSpec((BS, 1), lambda i: (i, 0)),
        out_shape=jax.ShapeDtypeStruct((B, 1), jnp.float32),
    )(x, lin_vals, a_mat, w1f, b1f, w2f, b2f, W3, c)
    return out


# speedup vs baseline: 1.4244x; 1.4244x over previous
import functools

import jax
import jax.numpy as jnp
from jax import lax
from jax.experimental import pallas as pl
from jax.experimental.pallas import tpu as pltpu
from jax.experimental.pallas import tpu_sc as plsc

NC, NS = 2, 16
NW = NC * NS
EPS = 1e-5


SC_V = 983040


def _sc_reformat(embT, v_pad):
    k, v = embT.shape
    vch = 1024
    per_w = SC_V // NW
    nch = per_w // vch
    out_rows = v_pad * k // 128
    orpc = vch * k // 128

    mesh = plsc.VectorSubcoreMesh(core_axis_name="c", subcore_axis_name="s")

    @functools.partial(
        pl.kernel,
        mesh=mesh,
        compiler_params=pltpu.CompilerParams(use_tc_tiling_on_sc=True,
                                             needs_layout_passes=False),
        out_type=jax.ShapeDtypeStruct((out_rows, 128), jnp.float32),
        scratch_types=[
            pltpu.VMEM((2, k, vch), jnp.float32),
            pltpu.VMEM((2, orpc, 128), jnp.float32),
            pltpu.SemaphoreType.DMA((2,)),
            pltpu.SemaphoreType.DMA((2,)),
        ],
    )
    def refmt_kernel(embT_hbm, out_hbm, slab, ob, isem, osem):
        wid = lax.axis_index("s") * NC + lax.axis_index("c")
        vbase = wid * per_w
        row_iota = lax.iota(jnp.int32, 16)
        in_pending = [None, None]
        out_pending = [None, None]
        in_pending[0] = pltpu.async_copy(
            embT_hbm.at[:, pl.ds(pl.multiple_of(vbase, 128), vch)],
            slab.at[0], isem.at[0])
        for ch in range(nch):
            cur = ch % 2
            nxt = (ch + 1) % 2
            if ch + 1 < nch:
                in_pending[nxt] = pltpu.async_copy(
                    embT_hbm.at[:, pl.ds(
                        pl.multiple_of(vbase + (ch + 1) * vch, 128), vch)],
                    slab.at[nxt], isem.at[nxt])
            in_pending[cur].wait()
            if out_pending[cur] is not None:
                out_pending[cur].wait()
            sl = slab.at[cur]
            obuf = ob.at[cur]

            @plsc.parallel_loop(0, vch // 8, unroll=8)
            def shuffle(jv, sl=sl, obuf=obuf):
                rowsplat = jnp.full((16,), jv, jnp.int32)
                colbase = row_iota * 0 + jv * 8
                for u in range(8):
                    col = colbase + u
                    val = plsc.load_gather(sl, [row_iota, col])
                    plsc.store_scatter(obuf, [rowsplat, row_iota + u * 16],
                                       val)
            out_pending[cur] = pltpu.async_copy(
                obuf,
                out_hbm.at[pl.ds(
                    pl.multiple_of((vbase + ch * vch) * k // 128, 128), orpc)],
                osem.at[cur])
        for p in out_pending:
            if p is not None:
                p.wait()

    return refmt_kernel(embT)


def _tc_tail_reformat(embT, bulk, v_pad):
    k, v = embT.shape
    c = 16384
    base_blk = SC_V // c
    g = pl.cdiv(v_pad - SC_V, c)
    out_rows = v_pad * k // 128
    rblk = c * k // 128

    def body(t_ref, al_ref, o_ref):
        x = t_ref[...]
        xt3 = x.T.reshape(c // 64, 64, k)
        out3 = jnp.concatenate(
            [xt3[:, 8 * m:8 * m + 8, :] for m in range(8)], axis=2)
        o_ref[...] = out3.reshape(rblk, 128)

    return pl.pallas_call(
        body,
        grid=(g,),
        in_specs=[pl.BlockSpec((k, c), lambda i: (0, i + base_blk)),
                  pl.BlockSpec((8, 128), lambda i: (0, 0))],
        out_specs=pl.BlockSpec((rblk, 128), lambda i: (i + base_blk, 0)),
        out_shape=jax.ShapeDtypeStruct((out_rows, 128), jnp.float32),
        input_output_aliases={1: 0},
    )(embT, bulk)


def _swap_idx(idx):
    lo = idx & 63
    return (idx ^ lo) | ((lo & 7) << 3) | (lo >> 3)


def _sc_gather(idx_flat, idx_raw, emb_table, lin_flat, *, n_idx, k):
    per_w = n_idx // NW
    chunk = 1664
    nchunk = per_w // chunk
    assert per_w % chunk == 0 and per_w % 8 == 0 and chunk % 8 == 0

    mesh = plsc.VectorSubcoreMesh(core_axis_name="c", subcore_axis_name="s")

    @functools.partial(
        pl.kernel,
        mesh=mesh,
        compiler_params=pltpu.CompilerParams(use_tc_tiling_on_sc=False),
        out_type=(
            jax.ShapeDtypeStruct((n_idx, k), jnp.float32),
            jax.ShapeDtypeStruct((n_idx,), jnp.float32),
        ),
        scratch_types=[
            pltpu.VMEM((per_w,), jnp.int32),
            pltpu.VMEM((per_w,), jnp.int32),
            pltpu.VMEM((2, chunk, k), jnp.float32),
            pltpu.VMEM((2, chunk), jnp.float32),
            pltpu.SemaphoreType.DMA((2,)),
            pltpu.SemaphoreType.DMA((2,)),
        ],
    )
    def gather_kernel(idx_hbm, idx2_hbm, emb_hbm, lin_hbm, erows_out,
                      lvals_out, idx_v, idx2_v, ebuf, lbuf, gsem, osem):
        wid = lax.axis_index("s") * NC + lax.axis_index("c")
        base = wid * per_w
        pltpu.sync_copy(idx_hbm.at[pl.ds(base, per_w)], idx_v)
        pltpu.sync_copy(idx2_hbm.at[pl.ds(base, per_w)], idx2_v)
        pending = [None, None]
        for g in range(nchunk):
            bsl = g % 2
            if pending[bsl] is not None:
                for h in pending[bsl]:
                    h.wait()
            isl = idx_v.at[pl.ds(g * chunk, chunk)]
            isl2 = idx2_v.at[pl.ds(g * chunk, chunk)]
            h_e = pltpu.async_copy(emb_hbm.at[isl], ebuf.at[bsl], gsem.at[bsl])
            h_l = pltpu.async_copy(lin_hbm.at[isl2], lbuf.at[bsl], gsem.at[bsl])
            h_e.wait()
            h_l.wait()
            o_e = pltpu.async_copy(
                ebuf.at[bsl], erows_out.at[pl.ds(base + g * chunk, chunk)],
                osem.at[bsl])
            o_l = pltpu.async_copy(
                lbuf.at[bsl], lvals_out.at[pl.ds(base + g * chunk, chunk)],
                osem.at[bsl])
            pending[bsl] = (o_e, o_l)
        for p in pending:
            if p is not None:
                for h in p:
                    h.wait()

    return gather_kernel(idx_flat, idx_raw, emb_table, lin_flat)


def _tc_body(x_ref, lin_ref, a_ref, w1_ref, b1_ref, w2_ref, b2_ref,
             w3_ref, c_ref, out_ref):
    x = x_ref[...]
    s = jnp.dot(x, a_ref[...], preferred_element_type=jnp.float32)
    fm = 0.5 * (jnp.sum(s * s, axis=1, keepdims=True)
                - jnp.sum(x * x, axis=1, keepdims=True))
    lin = jnp.sum(lin_ref[...], axis=1, keepdims=True)
    h = jnp.dot(x, w1_ref[...], preferred_element_type=jnp.float32) + b1_ref[...]
    h = jnp.maximum(h, 0.0)
    h = jnp.dot(h, w2_ref[...], preferred_element_type=jnp.float32) + b2_ref[...]
    h = jnp.maximum(h, 0.0)
    mlp = jnp.dot(h, w3_ref[...], preferred_element_type=jnp.float32)
    out_ref[...] = fm + lin + mlp + c_ref[...]


def kernel(cat_features, emb_table, lin_table, bias,
           W1, b1, g1, be1, W2, b2, g2, be2, W3, b3):
    B, F = cat_features.shape
    V, K = emb_table.shape
    D_IN = F * K
    H = W1.shape[1]
    n_idx = B * F

    idx_raw = cat_features.reshape(-1)
    idx_flat = jnp.where(idx_raw < SC_V, idx_raw, _swap_idx(idx_raw))

    v_pad = (V + 63) // 64 * 64
    bulk = _sc_reformat(emb_table.T, v_pad)
    emb2d = _tc_tail_reformat(emb_table.T, bulk, v_pad)
    lin1d = lin_table.reshape(-1)
    emb_lin = emb2d.reshape(v_pad, K)

    erows, lvals = _sc_gather(idx_flat, idx_raw, emb_lin, lin1d,
                              n_idx=n_idx, k=K)
    x = erows.reshape(B, D_IN)
    lin_vals = lvals.reshape(B, F)

    s_bn = 1.0 / jnp.sqrt(1.0 + EPS)
    w1f = W1 * (g1 * s_bn)[None, :]
    b1f = (b1 * g1 * s_bn + be1)[None, :]
    w2f = W2 * (g2 * s_bn)[None, :]
    b2f = (b2 * g2 * s_bn + be2)[None, :]
    c = (bias + b3).reshape(1, 1)
    a_mat = jnp.tile(jnp.eye(K, dtype=jnp.float32), (F, 1))

    BS = 1024
    grid = (B // BS,)
    out = pl.pallas_call(
        _tc_body,
        grid=grid,
        in_specs=[
            pl.BlockSpec((BS, D_IN), lambda i: (i, 0)),
            pl.BlockSpec((BS, F), lambda i: (i, 0)),
            pl.BlockSpec((D_IN, K), lambda i: (0, 0)),
            pl.BlockSpec((D_IN, H), lambda i: (0, 0)),
            pl.BlockSpec((1, H), lambda i: (0, 0)),
            pl.BlockSpec((H, H), lambda i: (0, 0)),
            pl.BlockSpec((1, H), lambda i: (0, 0)),
            pl.BlockSpec((H, 1), lambda i: (0, 0)),
            pl.BlockSpec((1, 1), lambda i: (0, 0)),
        ],
        out_specs=pl.BlockSpec((BS, 1), lambda i: (i, 0)),
        out_shape=jax.ShapeDtypeStruct((B, 1), jnp.float32),
    )(x, lin_vals, a_mat, w1f, b1f, w2f, b2f, W3, c)
    return out
